# baseline (speedup 1.0000x reference)
import functools

import jax
import jax.numpy as jnp
from jax import lax
from jax.experimental import pallas as pl
from jax.experimental.pallas import tpu as pltpu
from jax.experimental.pallas import tpu_sc as plsc

N = 10000
HN = N // 2
E = 160000
G = 64
F = 500
H = 64
PW = 2 * H
NPAD = 10240
NC, NS = 2, 16
EPAD = 163840
CPT = 40
CHUNK = 128
HB = 1000
GRID = HN // HB
NBUF = 8
ZR = 16
RPT = NPAD // NS

_f32 = jnp.float32
_HIGH = jax.lax.Precision.HIGHEST


def _dot(a, b):
    return jax.lax.dot_general(a, b, (((1,), (0,)), ((), ())),
                               precision=_HIGH, preferred_element_type=_f32)


def _dotT(a, b):
    return jax.lax.dot_general(a, b, (((0,), (0,)), ((), ())),
                               precision=_HIGH, preferred_element_type=_f32)


def _tc1_body(x_ref, wlt_ref, wrt_ref, z_ref, r_ref):
    outs = []
    for half in (0, 1):
        xb = x_ref[half]
        nrm = jnp.maximum(jnp.sum(jnp.abs(xb), axis=1, keepdims=True), 1e-12)
        xn = xb / nrm
        outs.append((_dot(xn, wlt_ref[...]), _dot(xn, wrt_ref[...])))
    z_ref[...] = jnp.concatenate([outs[0][0], outs[1][0]], axis=1)
    r_ref[...] = jnp.concatenate([outs[0][1], outs[1][1]], axis=1)


def _tc1(x, wlt, wrt):
    return pl.pallas_call(
        _tc1_body,
        grid=(GRID,),
        in_specs=[
            pl.BlockSpec((2, HB, F), lambda i: (0, i, 0)),
            pl.BlockSpec((F, H), lambda i: (0, 0)),
            pl.BlockSpec((F, H), lambda i: (0, 0)),
        ],
        out_specs=[
            pl.BlockSpec((HB, PW), lambda i: (i, 0)),
            pl.BlockSpec((HB, PW), lambda i: (i, 0)),
        ],
        out_shape=[
            jax.ShapeDtypeStruct((HN, PW), _f32),
            jax.ShapeDtypeStruct((HN, PW), _f32),
        ],
    )(x.reshape(2, HN, F), wlt, wrt)


_sc_mesh = dict(core_axis_name="c", subcore_axis_name="s",
                num_cores=NC, num_subcores=NS)


def _sc_zero_acc(sid, acc, zbuf):
    def zrow(i, c):
        for j in range(H // 16):
            zbuf[i, pl.ds(j * 16, 16)] = jnp.zeros((16,), _f32)
        return c
    lax.fori_loop(0, ZR, zrow, 0)

    def zcp(k, c):
        pltpu.sync_copy(zbuf, acc.at[pl.ds(sid * RPT + k * ZR, ZR)])
        return c
    lax.fori_loop(0, RPT // ZR, zcp, 0)


def _make_sc_scatter():
    mesh = plsc.VectorSubcoreMesh(**_sc_mesh)

    @functools.partial(
        pl.kernel,
        out_type=jax.ShapeDtypeStruct((NC * NPAD, H), _f32),
        mesh=mesh,
        compiler_params=pltpu.CompilerParams(use_tc_tiling_on_sc=False),
        scratch_types=[
            pltpu.VMEM_SHARED((NPAD, H), _f32),
            pltpu.VMEM((CPT, CHUNK), jnp.int32),
            pltpu.VMEM((CPT, CHUNK), jnp.int32),
            pltpu.VMEM((NBUF, CHUNK, H), _f32),
            pltpu.VMEM((ZR, H), _f32),
            [pltpu.SemaphoreType.DMA] * NBUF,
        ],
    )
    def sc_fn(z_hbm, src_hbm, dst_hbm, out_hbm, acc, idx_s, idx_d, rows,
              zbuf, sems):
        cid = lax.axis_index("c")
        sid = lax.axis_index("s")
        wid = cid * NS + sid
        _sc_zero_acc(sid, acc, zbuf)

        pltpu.sync_copy(src_hbm.at[pl.ds(wid * CPT, CPT)], idx_s)
        pltpu.sync_copy(dst_hbm.at[pl.ds(wid * CPT, CPT)], idx_d)
        plsc.subcore_barrier()

        def superchunk(s, carry):
            base = s * NBUF
            descs = [
                pltpu.async_copy(z_hbm.at[idx_s.at[base + b]], rows.at[b],
                                 sems[b])
                for b in range(NBUF)
            ]
            for b in range(NBUF):
                descs[b].wait()
                pltpu.sync_copy(rows.at[b], acc.at[idx_d.at[base + b]],
                                add=True)
            return carry
        lax.fori_loop(0, CPT // NBUF, superchunk, 0)
        plsc.subcore_barrier()

        pltpu.sync_copy(acc.at[pl.ds(sid * RPT, RPT)],
                        out_hbm.at[pl.ds(cid * NPAD + sid * RPT, RPT)])

    return sc_fn


def _make_sc_degree():
    mesh = plsc.VectorSubcoreMesh(**_sc_mesh)

    @functools.partial(
        pl.kernel,
        out_type=jax.ShapeDtypeStruct((NC * NPAD, H), _f32),
        mesh=mesh,
        compiler_params=pltpu.CompilerParams(use_tc_tiling_on_sc=False),
        scratch_types=[
            pltpu.VMEM_SHARED((NPAD, H), _f32),
            pltpu.VMEM((CPT, CHUNK), jnp.int32),
            pltpu.VMEM((CHUNK, H), _f32),
            pltpu.VMEM((ZR, H), _f32),
        ],
    )
    def deg_fn(dst_hbm, out_hbm, acc, idx_d, ones, zbuf):
        cid = lax.axis_index("c")
        sid = lax.axis_index("s")
        wid = cid * NS + sid
        _sc_zero_acc(sid, acc, zbuf)

        def orow(i, c):
            for j in range(H // 16):
                ones[i, pl.ds(j * 16, 16)] = jnp.ones((16,), _f32)
            return c
        lax.fori_loop(0, CHUNK, orow, 0)

        pltpu.sync_copy(dst_hbm.at[pl.ds(wid * CPT, CPT)], idx_d)
        plsc.subcore_barrier()

        def chunk(c, carry):
            pltpu.sync_copy(ones, acc.at[idx_d.at[c]], add=True)
            return carry
        lax.fori_loop(0, CPT, chunk, 0)
        plsc.subcore_barrier()

        pltpu.sync_copy(acc.at[pl.ds(sid * RPT, RPT)],
                        out_hbm.at[pl.ds(cid * NPAD + sid * RPT, RPT)])

    return deg_fn


_sc_cache = {}


def _sc_scatter_impl(zp, src2, dst2):
    if "scatter" not in _sc_cache:
        _sc_cache["scatter"] = _make_sc_scatter()
    out = _sc_cache["scatter"](zp.reshape(N, H), src2, dst2)
    return out.reshape(NC, NPAD // 2, PW)


def _sc_degree_impl(dst2):
    if "degree" not in _sc_cache:
        _sc_cache["degree"] = _make_sc_degree()
    return _sc_cache["degree"](dst2).reshape(NC, NPAD // 2, PW)


def _tc2_body(agg_ref, deg_ref, r_ref, b_ref, wl_ref, wr_ref, z_ref, r2_ref):
    s = agg_ref[0] + agg_ref[1]
    d = deg_ref[0] + deg_ref[1]
    e = s * (1.0 / jnp.maximum(d, 1.0)) + b_ref[...] + r_ref[...]
    z_ref[...] = _dot(e, wl_ref[...])
    r2_ref[...] = _dot(e, wr_ref[...])


def _tc2(agg, deg, r1, b, wlbd, wrbd):
    return pl.pallas_call(
        _tc2_body,
        grid=(GRID,),
        in_specs=[
            pl.BlockSpec((NC, HB, PW), lambda i: (0, i, 0)),
            pl.BlockSpec((NC, HB, PW), lambda i: (0, i, 0)),
            pl.BlockSpec((HB, PW), lambda i: (i, 0)),
            pl.BlockSpec((1, PW), lambda i: (0, 0)),
            pl.BlockSpec((PW, PW), lambda i: (0, 0)),
            pl.BlockSpec((PW, PW), lambda i: (0, 0)),
        ],
        out_specs=[
            pl.BlockSpec((HB, PW), lambda i: (i, 0)),
            pl.BlockSpec((HB, PW), lambda i: (i, 0)),
        ],
        out_shape=[
            jax.ShapeDtypeStruct((HN, PW), _f32),
            jax.ShapeDtypeStruct((HN, PW), _f32),
        ],
    )(agg, deg, r1, b, wlbd, wrbd)


def _tc4_body(agg_ref, deg_ref, r_ref, b_ref, ba_ref, bb_ref,
              l1w_ref, l1b_ref, l2w_ref, l2b_ref, l3w_ref, l3b_ref,
              l4w_ref, l4b_ref, out_ref, pooled, cnt):
    i = pl.program_id(0)

    @pl.when(i == 0)
    def _init():
        pooled[...] = jnp.zeros_like(pooled)
        cnt[...] = jnp.zeros_like(cnt)

    s = agg_ref[0] + agg_ref[1]
    d = deg_ref[0] + deg_ref[1]
    e3 = s * (1.0 / jnp.maximum(d, 1.0)) + b_ref[...] + r_ref[...]
    gid = lax.broadcasted_iota(jnp.int32, (HB, G), 1)
    oha = (ba_ref[0, 0, :][:, None] == gid).astype(_f32)
    ohb = (bb_ref[0, 0, :][:, None] == gid).astype(_f32)
    ones = jnp.ones((HB, 1), _f32)
    pooled[...] += _dotT(oha, e3[:, :H]) + _dotT(ohb, e3[:, H:])
    cnt[...] += _dotT(oha, ones) + _dotT(ohb, ones)

    @pl.when(i == GRID - 1)
    def _finish():
        c = pooled[...] * (1.0 / jnp.maximum(cnt[...], 1.0))
        h = jnp.tanh(_dot(c, l1w_ref[...]) + l1b_ref[...])
        h = jnp.tanh(_dot(h, l2w_ref[...]) + l2b_ref[...])
        h = jnp.tanh(_dot(h, l3w_ref[...]) + l3b_ref[...])
        out_ref[...] = _dot(h, l4w_ref[...]) + l4b_ref[...]


def _tc4(agg, deg, r3, b, batch_r, l1w, l1b, l2w, l2b, l3w, l3b, l4w, l4b):
    full = lambda a: pl.BlockSpec(a.shape, lambda i: tuple(0 for _ in a.shape))
    return pl.pallas_call(
        _tc4_body,
        grid=(GRID,),
        in_specs=[
            pl.BlockSpec((NC, HB, PW), lambda i: (0, i, 0)),
            pl.BlockSpec((NC, HB, PW), lambda i: (0, i, 0)),
            pl.BlockSpec((HB, PW), lambda i: (i, 0)),
            pl.BlockSpec((1, PW), lambda i: (0, 0)),
            pl.BlockSpec((1, 1, HB), lambda i: (i, 0, 0)),
            pl.BlockSpec((1, 1, HB), lambda i: (GRID + i, 0, 0)),
            full(l1w), full(l1b), full(l2w), full(l2b),
            full(l3w), full(l3b), full(l4w), full(l4b),
        ],
        out_specs=pl.BlockSpec((G, 80), lambda i: (0, 0)),
        out_shape=jax.ShapeDtypeStruct((G, 80), _f32),
        scratch_shapes=[
            pltpu.VMEM((G, H), _f32),
            pltpu.VMEM((G, 1), _f32),
        ],
    )(agg, deg, r3, b, batch_r, batch_r,
      l1w, l1b, l2w, l2b, l3w, l3b, l4w, l4b)


def kernel(x, edge_index, batch, y, W1l, b1l, W1r, W2l, b2l, W2r, W3l, b3l,
           W3r, lin1_W, lin1_b, bn1_g, bn1_b, bn1_m, bn1_v, lin2_W, lin2_b,
           bn2_g, bn2_b, bn2_m, bn2_v, lin3_W, lin3_b, bn3_g, bn3_b, bn3_m,
           bn3_v, lin4_W, lin4_b):
    rho = lambda v: jnp.where(v < HN, 2 * v, 2 * (v - HN) + 1)
    src = rho(edge_index[0])
    dst = rho(edge_index[1])
    pad = EPAD - E
    pad_i = jnp.arange(pad, dtype=jnp.int32)
    src2 = jnp.concatenate([src, pad_i % N]).reshape(EPAD // CHUNK, CHUNK)
    dst2 = jnp.concatenate([dst, N + pad_i % (NPAD - N)]).reshape(
        EPAD // CHUNK, CHUNK)
    batch_r = batch.reshape(2 * GRID, 1, HB)

    bd = lambda w: jnp.zeros((PW, PW), _f32).at[:H, :H].set(
        w.T).at[H:, H:].set(w.T)
    pk = lambda v: jnp.concatenate([v, v]).reshape(1, PW)
    row = lambda v: v.reshape(1, -1)

    def fold(Wt, b, g, bb, m, v):
        s = g / jnp.sqrt(v + 1e-5)
        return Wt * s[None, :], row(b * s + bb - m * s)

    l1w, l1b = fold(lin1_W.T, lin1_b, bn1_g, bn1_b, bn1_m, bn1_v)
    l2w, l2b = fold(lin2_W.T, lin2_b, bn2_g, bn2_b, bn2_m, bn2_v)
    l3w, l3b = fold(lin3_W.T, lin3_b, bn3_g, bn3_b, bn3_m, bn3_v)
    l4w, l4b = lin4_W.T, row(lin4_b)

    deg = _sc_degree_impl(dst2)
    z1, r1 = _tc1(x, W1l.T, W1r.T)
    agg1 = _sc_scatter_impl(z1, src2, dst2)
    z2, r2 = _tc2(agg1, deg, r1, pk(b1l), bd(W2l), bd(W2r))
    agg2 = _sc_scatter_impl(z2, src2, dst2)
    z3, r3 = _tc2(agg2, deg, r2, pk(b2l), bd(W3l), bd(W3r))
    agg3 = _sc_scatter_impl(z3, src2, dst2)
    return _tc4(agg3, deg, r3, pk(b3l), batch_r,
                l1w, l1b, l2w, l2b, l3w, l3b, l4w, l4b)


# speedup vs baseline: 1.1429x; 1.1429x over previous
import functools

import jax
import jax.numpy as jnp
from jax import lax
from jax.experimental import pallas as pl
from jax.experimental.pallas import tpu as pltpu
from jax.experimental.pallas import tpu_sc as plsc

N = 10000
HN = N // 2
E = 160000
G = 64
F = 500
H = 64
PW = 2 * H
NPAD = 10240
NC, NS = 2, 16
EPAD = 163840
CPT = 40
CHUNK = 128
HB = 1000
GRID = HN // HB
NBUF = 8
ZR = 16
RPT = NPAD // NS

_f32 = jnp.float32
_HIGH = jax.lax.Precision.HIGHEST


def _dot(a, b):
    return jax.lax.dot_general(a, b, (((1,), (0,)), ((), ())),
                               precision=_HIGH, preferred_element_type=_f32)


def _dotT(a, b):
    return jax.lax.dot_general(a, b, (((0,), (0,)), ((), ())),
                               precision=_HIGH, preferred_element_type=_f32)


def _tc1_body(xa_ref, xb_ref, wlt_ref, wrt_ref, z_ref, r_ref):
    outs = []
    for xref in (xa_ref, xb_ref):
        xb = xref[...]
        nrm = jnp.maximum(jnp.sum(jnp.abs(xb), axis=1, keepdims=True), 1e-12)
        xn = xb / nrm
        outs.append((_dot(xn, wlt_ref[...]), _dot(xn, wrt_ref[...])))
    z_ref[...] = jnp.concatenate([outs[0][0], outs[1][0]], axis=1)
    r_ref[...] = jnp.concatenate([outs[0][1], outs[1][1]], axis=1)


def _tc1(x, wlt, wrt):
    return pl.pallas_call(
        _tc1_body,
        grid=(GRID,),
        in_specs=[
            pl.BlockSpec((HB, F), lambda i: (i, 0)),
            pl.BlockSpec((HB, F), lambda i: (GRID + i, 0)),
            pl.BlockSpec((F, H), lambda i: (0, 0)),
            pl.BlockSpec((F, H), lambda i: (0, 0)),
        ],
        out_specs=[
            pl.BlockSpec((HB, PW), lambda i: (i, 0)),
            pl.BlockSpec((HB, PW), lambda i: (i, 0)),
        ],
        out_shape=[
            jax.ShapeDtypeStruct((HN, PW), _f32),
            jax.ShapeDtypeStruct((HN, PW), _f32),
        ],
    )(x, x, wlt, wrt)


_sc_mesh = dict(core_axis_name="c", subcore_axis_name="s",
                num_cores=NC, num_subcores=NS)


def _sc_zero_acc(sid, acc, zbuf):
    def zrow(i, c):
        for j in range(H // 16):
            zbuf[i, pl.ds(j * 16, 16)] = jnp.zeros((16,), _f32)
        return c
    lax.fori_loop(0, ZR, zrow, 0)

    def zcp(k, c):
        pltpu.sync_copy(zbuf, acc.at[pl.ds(sid * RPT + k * ZR, ZR)])
        return c
    lax.fori_loop(0, RPT // ZR, zcp, 0)


def _make_sc_scatter():
    mesh = plsc.VectorSubcoreMesh(**_sc_mesh)

    @functools.partial(
        pl.kernel,
        out_type=jax.ShapeDtypeStruct((NC * NPAD, H), _f32),
        mesh=mesh,
        compiler_params=pltpu.CompilerParams(use_tc_tiling_on_sc=False),
        scratch_types=[
            pltpu.VMEM_SHARED((NPAD, H), _f32),
            pltpu.VMEM((CPT, CHUNK), jnp.int32),
            pltpu.VMEM((CPT, CHUNK), jnp.int32),
            pltpu.VMEM((NBUF, CHUNK, H), _f32),
            pltpu.VMEM((ZR, H), _f32),
            [pltpu.SemaphoreType.DMA] * NBUF,
        ],
    )
    def sc_fn(z_hbm, src_hbm, dst_hbm, out_hbm, acc, idx_s, idx_d, rows,
              zbuf, sems):
        cid = lax.axis_index("c")
        sid = lax.axis_index("s")
        wid = cid * NS + sid
        _sc_zero_acc(sid, acc, zbuf)

        pltpu.sync_copy(src_hbm.at[pl.ds(wid * CPT, CPT)], idx_s)
        pltpu.sync_copy(dst_hbm.at[pl.ds(wid * CPT, CPT)], idx_d)
        plsc.subcore_barrier()

        def superchunk(s, carry):
            base = s * NBUF
            descs = [
                pltpu.async_copy(z_hbm.at[idx_s.at[base + b]], rows.at[b],
                                 sems[b])
                for b in range(NBUF)
            ]
            for b in range(NBUF):
                descs[b].wait()
                pltpu.sync_copy(rows.at[b], acc.at[idx_d.at[base + b]],
                                add=True)
            return carry
        lax.fori_loop(0, CPT // NBUF, superchunk, 0)
        plsc.subcore_barrier()

        pltpu.sync_copy(acc.at[pl.ds(sid * RPT, RPT)],
                        out_hbm.at[pl.ds(cid * NPAD + sid * RPT, RPT)])

    return sc_fn


def _make_sc_degree():
    mesh = plsc.VectorSubcoreMesh(**_sc_mesh)

    @functools.partial(
        pl.kernel,
        out_type=jax.ShapeDtypeStruct((NC * NPAD, H), _f32),
        mesh=mesh,
        compiler_params=pltpu.CompilerParams(use_tc_tiling_on_sc=False),
        scratch_types=[
            pltpu.VMEM_SHARED((NPAD, H), _f32),
            pltpu.VMEM((CPT, CHUNK), jnp.int32),
            pltpu.VMEM((CHUNK, H), _f32),
            pltpu.VMEM((ZR, H), _f32),
        ],
    )
    def deg_fn(dst_hbm, out_hbm, acc, idx_d, ones, zbuf):
        cid = lax.axis_index("c")
        sid = lax.axis_index("s")
        wid = cid * NS + sid
        _sc_zero_acc(sid, acc, zbuf)

        def orow(i, c):
            for j in range(H // 16):
                ones[i, pl.ds(j * 16, 16)] = jnp.ones((16,), _f32)
            return c
        lax.fori_loop(0, CHUNK, orow, 0)

        pltpu.sync_copy(dst_hbm.at[pl.ds(wid * CPT, CPT)], idx_d)
        plsc.subcore_barrier()

        def chunk(c, carry):
            pltpu.sync_copy(ones, acc.at[idx_d.at[c]], add=True)
            return carry
        lax.fori_loop(0, CPT, chunk, 0)
        plsc.subcore_barrier()

        pltpu.sync_copy(acc.at[pl.ds(sid * RPT, RPT)],
                        out_hbm.at[pl.ds(cid * NPAD + sid * RPT, RPT)])

    return deg_fn


_sc_cache = {}


def _sc_scatter_impl(zp, src2, dst2):
    if "scatter" not in _sc_cache:
        _sc_cache["scatter"] = _make_sc_scatter()
    out = _sc_cache["scatter"](zp.reshape(N, H), src2, dst2)
    return out.reshape(NC, NPAD // 2, PW)


def _sc_degree_impl(dst2):
    if "degree" not in _sc_cache:
        _sc_cache["degree"] = _make_sc_degree()
    return _sc_cache["degree"](dst2).reshape(NC, NPAD // 2, PW)


def _tc2_body(agg_ref, deg_ref, r_ref, b_ref, wl_ref, wr_ref, z_ref, r2_ref):
    s = agg_ref[0] + agg_ref[1]
    d = deg_ref[0] + deg_ref[1]
    e = s * (1.0 / jnp.maximum(d, 1.0)) + b_ref[...] + r_ref[...]
    z_ref[...] = _dot(e, wl_ref[...])
    r2_ref[...] = _dot(e, wr_ref[...])


def _tc2(agg, deg, r1, b, wlbd, wrbd):
    return pl.pallas_call(
        _tc2_body,
        grid=(GRID,),
        in_specs=[
            pl.BlockSpec((NC, HB, PW), lambda i: (0, i, 0)),
            pl.BlockSpec((NC, HB, PW), lambda i: (0, i, 0)),
            pl.BlockSpec((HB, PW), lambda i: (i, 0)),
            pl.BlockSpec((1, PW), lambda i: (0, 0)),
            pl.BlockSpec((PW, PW), lambda i: (0, 0)),
            pl.BlockSpec((PW, PW), lambda i: (0, 0)),
        ],
        out_specs=[
            pl.BlockSpec((HB, PW), lambda i: (i, 0)),
            pl.BlockSpec((HB, PW), lambda i: (i, 0)),
        ],
        out_shape=[
            jax.ShapeDtypeStruct((HN, PW), _f32),
            jax.ShapeDtypeStruct((HN, PW), _f32),
        ],
    )(agg, deg, r1, b, wlbd, wrbd)


def _tc4_body(agg_ref, deg_ref, r_ref, b_ref, ba_ref, bb_ref,
              l1w_ref, l1b_ref, l2w_ref, l2b_ref, l3w_ref, l3b_ref,
              l4w_ref, l4b_ref, out_ref, pooled, cnt):
    i = pl.program_id(0)

    @pl.when(i == 0)
    def _init():
        pooled[...] = jnp.zeros_like(pooled)
        cnt[...] = jnp.zeros_like(cnt)

    s = agg_ref[0] + agg_ref[1]
    d = deg_ref[0] + deg_ref[1]
    e3 = s * (1.0 / jnp.maximum(d, 1.0)) + b_ref[...] + r_ref[...]
    gid = lax.broadcasted_iota(jnp.int32, (HB, G), 1)
    oha = (ba_ref[0, 0, :][:, None] == gid).astype(_f32)
    ohb = (bb_ref[0, 0, :][:, None] == gid).astype(_f32)
    ones = jnp.ones((HB, 1), _f32)
    pooled[...] += _dotT(oha, e3[:, :H]) + _dotT(ohb, e3[:, H:])
    cnt[...] += _dotT(oha, ones) + _dotT(ohb, ones)

    @pl.when(i == GRID - 1)
    def _finish():
        c = pooled[...] * (1.0 / jnp.maximum(cnt[...], 1.0))
        h = jnp.tanh(_dot(c, l1w_ref[...]) + l1b_ref[...])
        h = jnp.tanh(_dot(h, l2w_ref[...]) + l2b_ref[...])
        h = jnp.tanh(_dot(h, l3w_ref[...]) + l3b_ref[...])
        out_ref[...] = _dot(h, l4w_ref[...]) + l4b_ref[...]


def _tc4(agg, deg, r3, b, batch_r, l1w, l1b, l2w, l2b, l3w, l3b, l4w, l4b):
    full = lambda a: pl.BlockSpec(a.shape, lambda i: tuple(0 for _ in a.shape))
    return pl.pallas_call(
        _tc4_body,
        grid=(GRID,),
        in_specs=[
            pl.BlockSpec((NC, HB, PW), lambda i: (0, i, 0)),
            pl.BlockSpec((NC, HB, PW), lambda i: (0, i, 0)),
            pl.BlockSpec((HB, PW), lambda i: (i, 0)),
            pl.BlockSpec((1, PW), lambda i: (0, 0)),
            pl.BlockSpec((1, 1, HB), lambda i: (i, 0, 0)),
            pl.BlockSpec((1, 1, HB), lambda i: (GRID + i, 0, 0)),
            full(l1w), full(l1b), full(l2w), full(l2b),
            full(l3w), full(l3b), full(l4w), full(l4b),
        ],
        out_specs=pl.BlockSpec((G, 80), lambda i: (0, 0)),
        out_shape=jax.ShapeDtypeStruct((G, 80), _f32),
        scratch_shapes=[
            pltpu.VMEM((G, H), _f32),
            pltpu.VMEM((G, 1), _f32),
        ],
    )(agg, deg, r3, b, batch_r, batch_r,
      l1w, l1b, l2w, l2b, l3w, l3b, l4w, l4b)


def kernel(x, edge_index, batch, y, W1l, b1l, W1r, W2l, b2l, W2r, W3l, b3l,
           W3r, lin1_W, lin1_b, bn1_g, bn1_b, bn1_m, bn1_v, lin2_W, lin2_b,
           bn2_g, bn2_b, bn2_m, bn2_v, lin3_W, lin3_b, bn3_g, bn3_b, bn3_m,
           bn3_v, lin4_W, lin4_b):
    rho = lambda v: jnp.where(v < HN, 2 * v, 2 * (v - HN) + 1)
    src = rho(edge_index[0])
    dst = rho(edge_index[1])
    pad = EPAD - E
    pad_i = jnp.arange(pad, dtype=jnp.int32)
    src2 = jnp.concatenate([src, pad_i % N]).reshape(EPAD // CHUNK, CHUNK)
    dst2 = jnp.concatenate([dst, N + pad_i % (NPAD - N)]).reshape(
        EPAD // CHUNK, CHUNK)
    batch_r = batch.reshape(2 * GRID, 1, HB)

    bd = lambda w: jnp.zeros((PW, PW), _f32).at[:H, :H].set(
        w.T).at[H:, H:].set(w.T)
    pk = lambda v: jnp.concatenate([v, v]).reshape(1, PW)
    row = lambda v: v.reshape(1, -1)

    def fold(Wt, b, g, bb, m, v):
        s = g / jnp.sqrt(v + 1e-5)
        return Wt * s[None, :], row(b * s + bb - m * s)

    l1w, l1b = fold(lin1_W.T, lin1_b, bn1_g, bn1_b, bn1_m, bn1_v)
    l2w, l2b = fold(lin2_W.T, lin2_b, bn2_g, bn2_b, bn2_m, bn2_v)
    l3w, l3b = fold(lin3_W.T, lin3_b, bn3_g, bn3_b, bn3_m, bn3_v)
    l4w, l4b = lin4_W.T, row(lin4_b)

    deg = _sc_degree_impl(dst2)
    z1, r1 = _tc1(x, W1l.T, W1r.T)
    agg1 = _sc_scatter_impl(z1, src2, dst2)
    z2, r2 = _tc2(agg1, deg, r1, pk(b1l), bd(W2l), bd(W2r))
    agg2 = _sc_scatter_impl(z2, src2, dst2)
    z3, r3 = _tc2(agg2, deg, r2, pk(b2l), bd(W3l), bd(W3r))
    agg3 = _sc_scatter_impl(z3, src2, dst2)
    return _tc4(agg3, deg, r3, pk(b3l), batch_r,
                l1w, l1b, l2w, l2b, l3w, l3b, l4w, l4b)


# speedup vs baseline: 1.2555x; 1.0985x over previous
import functools

import jax
import jax.numpy as jnp
from jax import lax
from jax.experimental import pallas as pl
from jax.experimental.pallas import tpu as pltpu
from jax.experimental.pallas import tpu_sc as plsc

N = 10000
HN = N // 2
E = 160000
G = 64
F = 500
H = 64
PW = 2 * H
NPAD = 10240
NC, NS = 2, 16
EPAD = 163840
CPT = 40
CHUNK = 128
HB = 1000
GRID = HN // HB
NBUF = 8
ZR = 16
RPT = NPAD // NS

_f32 = jnp.float32
_HIGH = jax.lax.Precision.HIGHEST


def _dot(a, b):
    return jax.lax.dot_general(a, b, (((1,), (0,)), ((), ())),
                               precision=_HIGH, preferred_element_type=_f32)


def _dotT(a, b):
    return jax.lax.dot_general(a, b, (((0,), (0,)), ((), ())),
                               precision=_HIGH, preferred_element_type=_f32)


def _dot_fast(a, b):
    return jax.lax.dot_general(a, b, (((1,), (0,)), ((), ())),
                               preferred_element_type=_f32)


def _tc1_body(xa_ref, xb_ref, wlt_ref, wrt_ref, z_ref, r_ref):
    outs = []
    for xref in (xa_ref, xb_ref):
        xb = xref[...]
        nrm = jnp.maximum(jnp.sum(jnp.abs(xb), axis=1, keepdims=True), 1e-12)
        xn = xb / nrm
        outs.append((_dot_fast(xn, wlt_ref[...]), _dot_fast(xn, wrt_ref[...])))
    z_ref[...] = jnp.concatenate([outs[0][0], outs[1][0]], axis=1)
    r_ref[...] = jnp.concatenate([outs[0][1], outs[1][1]], axis=1)


def _tc1(x, wlt, wrt):
    return pl.pallas_call(
        _tc1_body,
        grid=(GRID,),
        in_specs=[
            pl.BlockSpec((HB, F), lambda i: (i, 0)),
            pl.BlockSpec((HB, F), lambda i: (GRID + i, 0)),
            pl.BlockSpec((F, H), lambda i: (0, 0)),
            pl.BlockSpec((F, H), lambda i: (0, 0)),
        ],
        out_specs=[
            pl.BlockSpec((HB, PW), lambda i: (i, 0)),
            pl.BlockSpec((HB, PW), lambda i: (i, 0)),
        ],
        out_shape=[
            jax.ShapeDtypeStruct((HN, PW), _f32),
            jax.ShapeDtypeStruct((HN, PW), _f32),
        ],
    )(x, x, wlt, wrt)


_sc_mesh = dict(core_axis_name="c", subcore_axis_name="s",
                num_cores=NC, num_subcores=NS)


def _sc_zero_acc(sid, acc, zbuf):
    def zrow(i, c):
        for j in range(H // 16):
            zbuf[i, pl.ds(j * 16, 16)] = jnp.zeros((16,), _f32)
        return c
    lax.fori_loop(0, ZR, zrow, 0)

    def zcp(k, c):
        pltpu.sync_copy(zbuf, acc.at[pl.ds(sid * RPT + k * ZR, ZR)])
        return c
    lax.fori_loop(0, RPT // ZR, zcp, 0)


def _make_sc_scatter():
    mesh = plsc.VectorSubcoreMesh(**_sc_mesh)

    @functools.partial(
        pl.kernel,
        out_type=jax.ShapeDtypeStruct((NC * NPAD, H), _f32),
        mesh=mesh,
        compiler_params=pltpu.CompilerParams(use_tc_tiling_on_sc=False),
        scratch_types=[
            pltpu.VMEM_SHARED((NPAD, H), _f32),
            pltpu.VMEM((CPT, CHUNK), jnp.int32),
            pltpu.VMEM((CPT, CHUNK), jnp.int32),
            pltpu.VMEM((NBUF, CHUNK, H), _f32),
            pltpu.VMEM((ZR, H), _f32),
            [pltpu.SemaphoreType.DMA] * NBUF,
        ],
    )
    def sc_fn(z_hbm, src_hbm, dst_hbm, out_hbm, acc, idx_s, idx_d, rows,
              zbuf, sems):
        cid = lax.axis_index("c")
        sid = lax.axis_index("s")
        wid = cid * NS + sid
        _sc_zero_acc(sid, acc, zbuf)

        pltpu.sync_copy(src_hbm.at[pl.ds(wid * CPT, CPT)], idx_s)
        pltpu.sync_copy(dst_hbm.at[pl.ds(wid * CPT, CPT)], idx_d)
        plsc.subcore_barrier()

        def superchunk(s, carry):
            base = s * NBUF
            descs = [
                pltpu.async_copy(z_hbm.at[idx_s.at[base + b]], rows.at[b],
                                 sems[b])
                for b in range(NBUF)
            ]
            for b in range(NBUF):
                descs[b].wait()
                pltpu.sync_copy(rows.at[b], acc.at[idx_d.at[base + b]],
                                add=True)
            return carry
        lax.fori_loop(0, CPT // NBUF, superchunk, 0)
        plsc.subcore_barrier()

        pltpu.sync_copy(acc.at[pl.ds(sid * RPT, RPT)],
                        out_hbm.at[pl.ds(cid * NPAD + sid * RPT, RPT)])

    return sc_fn


def _make_sc_degree():
    mesh = plsc.VectorSubcoreMesh(**_sc_mesh)

    @functools.partial(
        pl.kernel,
        out_type=jax.ShapeDtypeStruct((NC * NPAD, H), _f32),
        mesh=mesh,
        compiler_params=pltpu.CompilerParams(use_tc_tiling_on_sc=False),
        scratch_types=[
            pltpu.VMEM_SHARED((NPAD, H), _f32),
            pltpu.VMEM((CPT, CHUNK), jnp.int32),
            pltpu.VMEM((CHUNK, H), _f32),
            pltpu.VMEM((ZR, H), _f32),
        ],
    )
    def deg_fn(dst_hbm, out_hbm, acc, idx_d, ones, zbuf):
        cid = lax.axis_index("c")
        sid = lax.axis_index("s")
        wid = cid * NS + sid
        _sc_zero_acc(sid, acc, zbuf)

        def orow(i, c):
            for j in range(H // 16):
                ones[i, pl.ds(j * 16, 16)] = jnp.ones((16,), _f32)
            return c
        lax.fori_loop(0, CHUNK, orow, 0)

        pltpu.sync_copy(dst_hbm.at[pl.ds(wid * CPT, CPT)], idx_d)
        plsc.subcore_barrier()

        def chunk(c, carry):
            pltpu.sync_copy(ones, acc.at[idx_d.at[c]], add=True)
            return carry
        lax.fori_loop(0, CPT, chunk, 0)
        plsc.subcore_barrier()

        pltpu.sync_copy(acc.at[pl.ds(sid * RPT, RPT)],
                        out_hbm.at[pl.ds(cid * NPAD + sid * RPT, RPT)])

    return deg_fn


_sc_cache = {}


def _sc_scatter_impl(zp, src2, dst2):
    if "scatter" not in _sc_cache:
        _sc_cache["scatter"] = _make_sc_scatter()
    out = _sc_cache["scatter"](zp.reshape(N, H), src2, dst2)
    return out.reshape(NC, NPAD // 2, PW)


def _sc_degree_impl(dst2):
    if "degree" not in _sc_cache:
        _sc_cache["degree"] = _make_sc_degree()
    return _sc_cache["degree"](dst2).reshape(NC, NPAD // 2, PW)


def _tc2_body(agg_ref, deg_ref, r_ref, b_ref, wl_ref, wr_ref, z_ref, r2_ref):
    s = agg_ref[0] + agg_ref[1]
    d = deg_ref[0] + deg_ref[1]
    e = s * (1.0 / jnp.maximum(d, 1.0)) + b_ref[...] + r_ref[...]
    z_ref[...] = _dot(e, wl_ref[...])
    r2_ref[...] = _dot(e, wr_ref[...])


def _tc2(agg, deg, r1, b, wlbd, wrbd):
    return pl.pallas_call(
        _tc2_body,
        grid=(GRID,),
        in_specs=[
            pl.BlockSpec((NC, HB, PW), lambda i: (0, i, 0)),
            pl.BlockSpec((NC, HB, PW), lambda i: (0, i, 0)),
            pl.BlockSpec((HB, PW), lambda i: (i, 0)),
            pl.BlockSpec((1, PW), lambda i: (0, 0)),
            pl.BlockSpec((PW, PW), lambda i: (0, 0)),
            pl.BlockSpec((PW, PW), lambda i: (0, 0)),
        ],
        out_specs=[
            pl.BlockSpec((HB, PW), lambda i: (i, 0)),
            pl.BlockSpec((HB, PW), lambda i: (i, 0)),
        ],
        out_shape=[
            jax.ShapeDtypeStruct((HN, PW), _f32),
            jax.ShapeDtypeStruct((HN, PW), _f32),
        ],
    )(agg, deg, r1, b, wlbd, wrbd)


def _tc4_body(agg_ref, deg_ref, r_ref, b_ref, ba_ref, bb_ref,
              l1w_ref, l1b_ref, l2w_ref, l2b_ref, l3w_ref, l3b_ref,
              l4w_ref, l4b_ref, out_ref, pooled, cnt):
    i = pl.program_id(0)

    @pl.when(i == 0)
    def _init():
        pooled[...] = jnp.zeros_like(pooled)
        cnt[...] = jnp.zeros_like(cnt)

    s = agg_ref[0] + agg_ref[1]
    d = deg_ref[0] + deg_ref[1]
    e3 = s * (1.0 / jnp.maximum(d, 1.0)) + b_ref[...] + r_ref[...]
    gid = lax.broadcasted_iota(jnp.int32, (HB, G), 1)
    oha = (ba_ref[0, 0, :][:, None] == gid).astype(_f32)
    ohb = (bb_ref[0, 0, :][:, None] == gid).astype(_f32)
    ones = jnp.ones((HB, 1), _f32)
    pooled[...] += _dotT(oha, e3[:, :H]) + _dotT(ohb, e3[:, H:])
    cnt[...] += _dotT(oha, ones) + _dotT(ohb, ones)

    @pl.when(i == GRID - 1)
    def _finish():
        c = pooled[...] * (1.0 / jnp.maximum(cnt[...], 1.0))
        h = jnp.tanh(_dot(c, l1w_ref[...]) + l1b_ref[...])
        h = jnp.tanh(_dot(h, l2w_ref[...]) + l2b_ref[...])
        h = jnp.tanh(_dot(h, l3w_ref[...]) + l3b_ref[...])
        out_ref[...] = _dot(h, l4w_ref[...]) + l4b_ref[...]


def _tc4(agg, deg, r3, b, batch_r, l1w, l1b, l2w, l2b, l3w, l3b, l4w, l4b):
    full = lambda a: pl.BlockSpec(a.shape, lambda i: tuple(0 for _ in a.shape))
    return pl.pallas_call(
        _tc4_body,
        grid=(GRID,),
        in_specs=[
            pl.BlockSpec((NC, HB, PW), lambda i: (0, i, 0)),
            pl.BlockSpec((NC, HB, PW), lambda i: (0, i, 0)),
            pl.BlockSpec((HB, PW), lambda i: (i, 0)),
            pl.BlockSpec((1, PW), lambda i: (0, 0)),
            pl.BlockSpec((1, 1, HB), lambda i: (i, 0, 0)),
            pl.BlockSpec((1, 1, HB), lambda i: (GRID + i, 0, 0)),
            full(l1w), full(l1b), full(l2w), full(l2b),
            full(l3w), full(l3b), full(l4w), full(l4b),
        ],
        out_specs=pl.BlockSpec((G, 80), lambda i: (0, 0)),
        out_shape=jax.ShapeDtypeStruct((G, 80), _f32),
        scratch_shapes=[
            pltpu.VMEM((G, H), _f32),
            pltpu.VMEM((G, 1), _f32),
        ],
    )(agg, deg, r3, b, batch_r, batch_r,
      l1w, l1b, l2w, l2b, l3w, l3b, l4w, l4b)


def kernel(x, edge_index, batch, y, W1l, b1l, W1r, W2l, b2l, W2r, W3l, b3l,
           W3r, lin1_W, lin1_b, bn1_g, bn1_b, bn1_m, bn1_v, lin2_W, lin2_b,
           bn2_g, bn2_b, bn2_m, bn2_v, lin3_W, lin3_b, bn3_g, bn3_b, bn3_m,
           bn3_v, lin4_W, lin4_b):
    rho = lambda v: jnp.where(v < HN, 2 * v, 2 * (v - HN) + 1)
    src = rho(edge_index[0])
    dst = rho(edge_index[1])
    pad = EPAD - E
    pad_i = jnp.arange(pad, dtype=jnp.int32)
    src2 = jnp.concatenate([src, pad_i % N]).reshape(EPAD // CHUNK, CHUNK)
    dst2 = jnp.concatenate([dst, N + pad_i % (NPAD - N)]).reshape(
        EPAD // CHUNK, CHUNK)
    batch_r = batch.reshape(2 * GRID, 1, HB)

    bd = lambda w: jnp.zeros((PW, PW), _f32).at[:H, :H].set(
        w.T).at[H:, H:].set(w.T)
    pk = lambda v: jnp.concatenate([v, v]).reshape(1, PW)
    row = lambda v: v.reshape(1, -1)

    def fold(Wt, b, g, bb, m, v):
        s = g / jnp.sqrt(v + 1e-5)
        return Wt * s[None, :], row(b * s + bb - m * s)

    l1w, l1b = fold(lin1_W.T, lin1_b, bn1_g, bn1_b, bn1_m, bn1_v)
    l2w, l2b = fold(lin2_W.T, lin2_b, bn2_g, bn2_b, bn2_m, bn2_v)
    l3w, l3b = fold(lin3_W.T, lin3_b, bn3_g, bn3_b, bn3_m, bn3_v)
    l4w, l4b = lin4_W.T, row(lin4_b)

    deg = _sc_degree_impl(dst2)
    z1, r1 = _tc1(x, W1l.T, W1r.T)
    agg1 = _sc_scatter_impl(z1, src2, dst2)
    z2, r2 = _tc2(agg1, deg, r1, pk(b1l), bd(W2l), bd(W2r))
    agg2 = _sc_scatter_impl(z2, src2, dst2)
    z3, r3 = _tc2(agg2, deg, r2, pk(b2l), bd(W3l), bd(W3r))
    agg3 = _sc_scatter_impl(z3, src2, dst2)
    return _tc4(agg3, deg, r3, pk(b3l), batch_r,
                l1w, l1b, l2w, l2b, l3w, l3b, l4w, l4b)


# speedup vs baseline: 1.2902x; 1.0276x over previous
import functools

import jax
import jax.numpy as jnp
from jax import lax
from jax.experimental import pallas as pl
from jax.experimental.pallas import tpu as pltpu
from jax.experimental.pallas import tpu_sc as plsc

N = 10000
HN = N // 2
E = 160000
G = 64
F = 500
H = 64
PW = 2 * H
NPAD = 10240
NC, NS = 2, 16
EPAD = 163840
CPT = 40
CHUNK = 128
HB = 1000
GRID = HN // HB
NBUF = 8
ZR = 16
RPT = NPAD // NS

_f32 = jnp.float32
_HIGH = jax.lax.Precision.HIGHEST


def _dot(a, b):
    return jax.lax.dot_general(a, b, (((1,), (0,)), ((), ())),
                               precision=_HIGH, preferred_element_type=_f32)


def _dotT(a, b):
    return jax.lax.dot_general(a, b, (((0,), (0,)), ((), ())),
                               precision=_HIGH, preferred_element_type=_f32)


def _dot_fast(a, b):
    return jax.lax.dot_general(a, b, (((1,), (0,)), ((), ())),
                               preferred_element_type=_f32)


def _tc1_body(xa_ref, xb_ref, wlt_ref, wrt_ref, z_ref, r_ref):
    outs = []
    for xref in (xa_ref, xb_ref):
        xb = xref[...]
        nrm = jnp.maximum(jnp.sum(jnp.abs(xb), axis=1, keepdims=True), 1e-12)
        xn = xb / nrm
        outs.append((_dot_fast(xn, wlt_ref[...]), _dot_fast(xn, wrt_ref[...])))
    z_ref[...] = jnp.concatenate([outs[0][0], outs[1][0]], axis=1)
    r_ref[...] = jnp.concatenate([outs[0][1], outs[1][1]], axis=1)


def _tc1(x, wlt, wrt):
    return pl.pallas_call(
        _tc1_body,
        grid=(GRID,),
        in_specs=[
            pl.BlockSpec((HB, F), lambda i: (i, 0)),
            pl.BlockSpec((HB, F), lambda i: (GRID + i, 0)),
            pl.BlockSpec((F, H), lambda i: (0, 0)),
            pl.BlockSpec((F, H), lambda i: (0, 0)),
        ],
        out_specs=[
            pl.BlockSpec((HB, PW), lambda i: (i, 0)),
            pl.BlockSpec((HB, PW), lambda i: (i, 0)),
        ],
        out_shape=[
            jax.ShapeDtypeStruct((HN, PW), _f32),
            jax.ShapeDtypeStruct((HN, PW), _f32),
        ],
    )(x, x, wlt, wrt)


_sc_mesh = dict(core_axis_name="c", subcore_axis_name="s",
                num_cores=NC, num_subcores=NS)


def _sc_zero_acc(sid, acc, zbuf):
    def zrow(i, c):
        for j in range(H // 16):
            zbuf[i, pl.ds(j * 16, 16)] = jnp.zeros((16,), _f32)
        return c
    lax.fori_loop(0, ZR, zrow, 0)

    def zcp(k, c):
        pltpu.sync_copy(zbuf, acc.at[pl.ds(sid * RPT + k * ZR, ZR)])
        return c
    lax.fori_loop(0, RPT // ZR, zcp, 0)


def _make_sc_scatter():
    mesh = plsc.VectorSubcoreMesh(**_sc_mesh)

    @functools.partial(
        pl.kernel,
        out_type=jax.ShapeDtypeStruct((NC * NPAD, H), _f32),
        mesh=mesh,
        compiler_params=pltpu.CompilerParams(use_tc_tiling_on_sc=False),
        scratch_types=[
            pltpu.VMEM_SHARED((NPAD, H), _f32),
            pltpu.VMEM((CPT, CHUNK), jnp.int32),
            pltpu.VMEM((CPT, CHUNK), jnp.int32),
            pltpu.VMEM((NBUF, CHUNK, H), _f32),
            pltpu.VMEM((ZR, H), _f32),
            [pltpu.SemaphoreType.DMA] * NBUF,
        ],
    )
    def sc_fn(z_hbm, src_hbm, dst_hbm, out_hbm, acc, idx_s, idx_d, rows,
              zbuf, sems):
        cid = lax.axis_index("c")
        sid = lax.axis_index("s")
        wid = cid * NS + sid
        _sc_zero_acc(sid, acc, zbuf)

        pltpu.sync_copy(src_hbm.at[pl.ds(wid * CPT, CPT)], idx_s)
        pltpu.sync_copy(dst_hbm.at[pl.ds(wid * CPT, CPT)], idx_d)
        plsc.subcore_barrier()

        def superchunk(s, carry):
            base = s * NBUF
            descs = [
                pltpu.async_copy(z_hbm.at[idx_s.at[base + b]], rows.at[b],
                                 sems[b])
                for b in range(NBUF)
            ]
            for b in range(NBUF):
                descs[b].wait()
                pltpu.sync_copy(rows.at[b], acc.at[idx_d.at[base + b]],
                                add=True)
            return carry
        lax.fori_loop(0, CPT // NBUF, superchunk, 0)
        plsc.subcore_barrier()

        pltpu.sync_copy(acc.at[pl.ds(sid * RPT, RPT)],
                        out_hbm.at[pl.ds(cid * NPAD + sid * RPT, RPT)])

    return sc_fn


def _make_sc_degree():
    mesh = plsc.VectorSubcoreMesh(**_sc_mesh)

    @functools.partial(
        pl.kernel,
        out_type=jax.ShapeDtypeStruct((NC * NPAD, H), _f32),
        mesh=mesh,
        compiler_params=pltpu.CompilerParams(use_tc_tiling_on_sc=False),
        scratch_types=[
            pltpu.VMEM_SHARED((NPAD, H), _f32),
            pltpu.VMEM((CPT, CHUNK), jnp.int32),
            pltpu.VMEM((CHUNK, H), _f32),
            pltpu.VMEM((ZR, H), _f32),
        ],
    )
    def deg_fn(dst_hbm, out_hbm, acc, idx_d, ones, zbuf):
        cid = lax.axis_index("c")
        sid = lax.axis_index("s")
        wid = cid * NS + sid
        _sc_zero_acc(sid, acc, zbuf)

        def orow(i, c):
            for j in range(H // 16):
                ones[i, pl.ds(j * 16, 16)] = jnp.ones((16,), _f32)
            return c
        lax.fori_loop(0, CHUNK, orow, 0)

        pltpu.sync_copy(dst_hbm.at[pl.ds(wid * CPT, CPT)], idx_d)
        plsc.subcore_barrier()

        def chunk(c, carry):
            pltpu.sync_copy(ones, acc.at[idx_d.at[c]], add=True)
            return carry
        lax.fori_loop(0, CPT, chunk, 0)
        plsc.subcore_barrier()

        pltpu.sync_copy(acc.at[pl.ds(sid * RPT, RPT)],
                        out_hbm.at[pl.ds(cid * NPAD + sid * RPT, RPT)])

    return deg_fn


_sc_cache = {}


def _sc_scatter_impl(zp, src2, dst2):
    if "scatter" not in _sc_cache:
        _sc_cache["scatter"] = _make_sc_scatter()
    out = _sc_cache["scatter"](zp.reshape(N, H), src2, dst2)
    return out.reshape(NC, NPAD // 2, PW)


def _sc_degree_impl(dst2):
    if "degree" not in _sc_cache:
        _sc_cache["degree"] = _make_sc_degree()
    return _sc_cache["degree"](dst2).reshape(NC, NPAD // 2, PW)


def _tc2_body(agg_ref, deg_ref, r_ref, b_ref, wl_ref, wr_ref, z_ref, r2_ref):
    s = agg_ref[0] + agg_ref[1]
    d = deg_ref[0] + deg_ref[1]
    e = s * (1.0 / jnp.maximum(d, 1.0)) + b_ref[...] + r_ref[...]
    z_ref[...] = _dot_fast(e, wl_ref[...])
    r2_ref[...] = _dot_fast(e, wr_ref[...])


def _tc2(agg, deg, r1, b, wlbd, wrbd):
    return pl.pallas_call(
        _tc2_body,
        grid=(GRID,),
        in_specs=[
            pl.BlockSpec((NC, HB, PW), lambda i: (0, i, 0)),
            pl.BlockSpec((NC, HB, PW), lambda i: (0, i, 0)),
            pl.BlockSpec((HB, PW), lambda i: (i, 0)),
            pl.BlockSpec((1, PW), lambda i: (0, 0)),
            pl.BlockSpec((PW, PW), lambda i: (0, 0)),
            pl.BlockSpec((PW, PW), lambda i: (0, 0)),
        ],
        out_specs=[
            pl.BlockSpec((HB, PW), lambda i: (i, 0)),
            pl.BlockSpec((HB, PW), lambda i: (i, 0)),
        ],
        out_shape=[
            jax.ShapeDtypeStruct((HN, PW), _f32),
            jax.ShapeDtypeStruct((HN, PW), _f32),
        ],
    )(agg, deg, r1, b, wlbd, wrbd)


def _tc4_body(agg_ref, deg_ref, r_ref, b_ref, ba_ref, bb_ref,
              l1w_ref, l1b_ref, l2w_ref, l2b_ref, l3w_ref, l3b_ref,
              l4w_ref, l4b_ref, out_ref, pooled, cnt):
    i = pl.program_id(0)

    @pl.when(i == 0)
    def _init():
        pooled[...] = jnp.zeros_like(pooled)
        cnt[...] = jnp.zeros_like(cnt)

    s = agg_ref[0] + agg_ref[1]
    d = deg_ref[0] + deg_ref[1]
    e3 = s * (1.0 / jnp.maximum(d, 1.0)) + b_ref[...] + r_ref[...]
    gid = lax.broadcasted_iota(jnp.int32, (HB, G), 1)
    oha = (ba_ref[0, 0, :][:, None] == gid).astype(_f32)
    ohb = (bb_ref[0, 0, :][:, None] == gid).astype(_f32)
    ones = jnp.ones((HB, 1), _f32)
    pooled[...] += _dotT(oha, e3[:, :H]) + _dotT(ohb, e3[:, H:])
    cnt[...] += _dotT(oha, ones) + _dotT(ohb, ones)

    @pl.when(i == GRID - 1)
    def _finish():
        c = pooled[...] * (1.0 / jnp.maximum(cnt[...], 1.0))
        h = jnp.tanh(_dot(c, l1w_ref[...]) + l1b_ref[...])
        h = jnp.tanh(_dot(h, l2w_ref[...]) + l2b_ref[...])
        h = jnp.tanh(_dot(h, l3w_ref[...]) + l3b_ref[...])
        out_ref[...] = _dot(h, l4w_ref[...]) + l4b_ref[...]


def _tc4(agg, deg, r3, b, batch_r, l1w, l1b, l2w, l2b, l3w, l3b, l4w, l4b):
    full = lambda a: pl.BlockSpec(a.shape, lambda i: tuple(0 for _ in a.shape))
    return pl.pallas_call(
        _tc4_body,
        grid=(GRID,),
        in_specs=[
            pl.BlockSpec((NC, HB, PW), lambda i: (0, i, 0)),
            pl.BlockSpec((NC, HB, PW), lambda i: (0, i, 0)),
            pl.BlockSpec((HB, PW), lambda i: (i, 0)),
            pl.BlockSpec((1, PW), lambda i: (0, 0)),
            pl.BlockSpec((1, 1, HB), lambda i: (i, 0, 0)),
            pl.BlockSpec((1, 1, HB), lambda i: (GRID + i, 0, 0)),
            full(l1w), full(l1b), full(l2w), full(l2b),
            full(l3w), full(l3b), full(l4w), full(l4b),
        ],
        out_specs=pl.BlockSpec((G, 80), lambda i: (0, 0)),
        out_shape=jax.ShapeDtypeStruct((G, 80), _f32),
        scratch_shapes=[
            pltpu.VMEM((G, H), _f32),
            pltpu.VMEM((G, 1), _f32),
        ],
    )(agg, deg, r3, b, batch_r, batch_r,
      l1w, l1b, l2w, l2b, l3w, l3b, l4w, l4b)


def kernel(x, edge_index, batch, y, W1l, b1l, W1r, W2l, b2l, W2r, W3l, b3l,
           W3r, lin1_W, lin1_b, bn1_g, bn1_b, bn1_m, bn1_v, lin2_W, lin2_b,
           bn2_g, bn2_b, bn2_m, bn2_v, lin3_W, lin3_b, bn3_g, bn3_b, bn3_m,
           bn3_v, lin4_W, lin4_b):
    rho = lambda v: jnp.where(v < HN, 2 * v, 2 * (v - HN) + 1)
    src = rho(edge_index[0])
    dst = rho(edge_index[1])
    pad = EPAD - E
    pad_i = jnp.arange(pad, dtype=jnp.int32)
    src2 = jnp.concatenate([src, pad_i % N]).reshape(EPAD // CHUNK, CHUNK)
    dst2 = jnp.concatenate([dst, N + pad_i % (NPAD - N)]).reshape(
        EPAD // CHUNK, CHUNK)
    batch_r = batch.reshape(2 * GRID, 1, HB)

    bd = lambda w: jnp.zeros((PW, PW), _f32).at[:H, :H].set(
        w.T).at[H:, H:].set(w.T)
    pk = lambda v: jnp.concatenate([v, v]).reshape(1, PW)
    row = lambda v: v.reshape(1, -1)

    def fold(Wt, b, g, bb, m, v):
        s = g / jnp.sqrt(v + 1e-5)
        return Wt * s[None, :], row(b * s + bb - m * s)

    l1w, l1b = fold(lin1_W.T, lin1_b, bn1_g, bn1_b, bn1_m, bn1_v)
    l2w, l2b = fold(lin2_W.T, lin2_b, bn2_g, bn2_b, bn2_m, bn2_v)
    l3w, l3b = fold(lin3_W.T, lin3_b, bn3_g, bn3_b, bn3_m, bn3_v)
    l4w, l4b = lin4_W.T, row(lin4_b)

    deg = _sc_degree_impl(dst2)
    z1, r1 = _tc1(x, W1l.T, W1r.T)
    agg1 = _sc_scatter_impl(z1, src2, dst2)
    z2, r2 = _tc2(agg1, deg, r1, pk(b1l), bd(W2l), bd(W2r))
    agg2 = _sc_scatter_impl(z2, src2, dst2)
    z3, r3 = _tc2(agg2, deg, r2, pk(b2l), bd(W3l), bd(W3r))
    agg3 = _sc_scatter_impl(z3, src2, dst2)
    return _tc4(agg3, deg, r3, pk(b3l), batch_r,
                l1w, l1b, l2w, l2b, l3w, l3b, l4w, l4b)


# speedup vs baseline: 1.2965x; 1.0049x over previous
import functools

import jax
import jax.numpy as jnp
from jax import lax
from jax.experimental import pallas as pl
from jax.experimental.pallas import tpu as pltpu
from jax.experimental.pallas import tpu_sc as plsc

N = 10000
HN = N // 2
E = 160000
G = 64
F = 500
H = 64
PW = 2 * H
NPAD = 10240
NC, NS = 2, 16
EPAD = 163840
CPT = 40
CHUNK = 128
HB = 1000
GRID = HN // HB
NBUF = 8
ZR = 16
RPT = NPAD // NS

_f32 = jnp.float32
_HIGH = jax.lax.Precision.HIGHEST


def _dot(a, b):
    return jax.lax.dot_general(a, b, (((1,), (0,)), ((), ())),
                               precision=_HIGH, preferred_element_type=_f32)


def _dotT(a, b):
    return jax.lax.dot_general(a, b, (((0,), (0,)), ((), ())),
                               preferred_element_type=_f32)


def _dot_fast(a, b):
    return jax.lax.dot_general(a, b, (((1,), (0,)), ((), ())),
                               preferred_element_type=_f32)


def _tc1_body(xa_ref, xb_ref, wlt_ref, wrt_ref, z_ref, r_ref):
    outs = []
    for xref in (xa_ref, xb_ref):
        xb = xref[...]
        nrm = jnp.maximum(jnp.sum(jnp.abs(xb), axis=1, keepdims=True), 1e-12)
        xn = xb / nrm
        outs.append((_dot_fast(xn, wlt_ref[...]), _dot_fast(xn, wrt_ref[...])))
    z_ref[...] = jnp.concatenate([outs[0][0], outs[1][0]], axis=1)
    r_ref[...] = jnp.concatenate([outs[0][1], outs[1][1]], axis=1)


def _tc1(x, wlt, wrt):
    return pl.pallas_call(
        _tc1_body,
        grid=(GRID,),
        in_specs=[
            pl.BlockSpec((HB, F), lambda i: (i, 0)),
            pl.BlockSpec((HB, F), lambda i: (GRID + i, 0)),
            pl.BlockSpec((F, H), lambda i: (0, 0)),
            pl.BlockSpec((F, H), lambda i: (0, 0)),
        ],
        out_specs=[
            pl.BlockSpec((HB, PW), lambda i: (i, 0)),
            pl.BlockSpec((HB, PW), lambda i: (i, 0)),
        ],
        out_shape=[
            jax.ShapeDtypeStruct((HN, PW), _f32),
            jax.ShapeDtypeStruct((HN, PW), _f32),
        ],
    )(x, x, wlt, wrt)


_sc_mesh = dict(core_axis_name="c", subcore_axis_name="s",
                num_cores=NC, num_subcores=NS)


def _sc_zero_acc(sid, acc, zbuf):
    def zrow(i, c):
        for j in range(H // 16):
            zbuf[i, pl.ds(j * 16, 16)] = jnp.zeros((16,), _f32)
        return c
    lax.fori_loop(0, ZR, zrow, 0)

    def zcp(k, c):
        pltpu.sync_copy(zbuf, acc.at[pl.ds(sid * RPT + k * ZR, ZR)])
        return c
    lax.fori_loop(0, RPT // ZR, zcp, 0)


def _make_sc_scatter():
    mesh = plsc.VectorSubcoreMesh(**_sc_mesh)

    @functools.partial(
        pl.kernel,
        out_type=jax.ShapeDtypeStruct((NC * NPAD, H), _f32),
        mesh=mesh,
        compiler_params=pltpu.CompilerParams(use_tc_tiling_on_sc=False),
        scratch_types=[
            pltpu.VMEM_SHARED((NPAD, H), _f32),
            pltpu.VMEM((CPT, CHUNK), jnp.int32),
            pltpu.VMEM((CPT, CHUNK), jnp.int32),
            pltpu.VMEM((NBUF, CHUNK, H), _f32),
            pltpu.VMEM((ZR, H), _f32),
            [pltpu.SemaphoreType.DMA] * NBUF,
        ],
    )
    def sc_fn(z_hbm, src_hbm, dst_hbm, out_hbm, acc, idx_s, idx_d, rows,
              zbuf, sems):
        cid = lax.axis_index("c")
        sid = lax.axis_index("s")
        wid = cid * NS + sid
        _sc_zero_acc(sid, acc, zbuf)

        pltpu.sync_copy(src_hbm.at[pl.ds(wid * CPT, CPT)], idx_s)
        pltpu.sync_copy(dst_hbm.at[pl.ds(wid * CPT, CPT)], idx_d)
        plsc.subcore_barrier()

        def superchunk(s, carry):
            base = s * NBUF
            descs = [
                pltpu.async_copy(z_hbm.at[idx_s.at[base + b]], rows.at[b],
                                 sems[b])
                for b in range(NBUF)
            ]
            for b in range(NBUF):
                descs[b].wait()
                pltpu.sync_copy(rows.at[b], acc.at[idx_d.at[base + b]],
                                add=True)
            return carry
        lax.fori_loop(0, CPT // NBUF, superchunk, 0)
        plsc.subcore_barrier()

        pltpu.sync_copy(acc.at[pl.ds(sid * RPT, RPT)],
                        out_hbm.at[pl.ds(cid * NPAD + sid * RPT, RPT)])

    return sc_fn


def _make_sc_degree():
    mesh = plsc.VectorSubcoreMesh(**_sc_mesh)

    @functools.partial(
        pl.kernel,
        out_type=jax.ShapeDtypeStruct((NC * NPAD, H), _f32),
        mesh=mesh,
        compiler_params=pltpu.CompilerParams(use_tc_tiling_on_sc=False),
        scratch_types=[
            pltpu.VMEM_SHARED((NPAD, H), _f32),
            pltpu.VMEM((CPT, CHUNK), jnp.int32),
            pltpu.VMEM((CHUNK, H), _f32),
            pltpu.VMEM((ZR, H), _f32),
        ],
    )
    def deg_fn(dst_hbm, out_hbm, acc, idx_d, ones, zbuf):
        cid = lax.axis_index("c")
        sid = lax.axis_index("s")
        wid = cid * NS + sid
        _sc_zero_acc(sid, acc, zbuf)

        def orow(i, c):
            for j in range(H // 16):
                ones[i, pl.ds(j * 16, 16)] = jnp.ones((16,), _f32)
            return c
        lax.fori_loop(0, CHUNK, orow, 0)

        pltpu.sync_copy(dst_hbm.at[pl.ds(wid * CPT, CPT)], idx_d)
        plsc.subcore_barrier()

        def chunk(c, carry):
            pltpu.sync_copy(ones, acc.at[idx_d.at[c]], add=True)
            return carry
        lax.fori_loop(0, CPT, chunk, 0)
        plsc.subcore_barrier()

        pltpu.sync_copy(acc.at[pl.ds(sid * RPT, RPT)],
                        out_hbm.at[pl.ds(cid * NPAD + sid * RPT, RPT)])

    return deg_fn


_sc_cache = {}


def _sc_scatter_impl(zp, src2, dst2):
    if "scatter" not in _sc_cache:
        _sc_cache["scatter"] = _make_sc_scatter()
    out = _sc_cache["scatter"](zp.reshape(N, H), src2, dst2)
    return out.reshape(NC, NPAD // 2, PW)


def _sc_degree_impl(dst2):
    if "degree" not in _sc_cache:
        _sc_cache["degree"] = _make_sc_degree()
    return _sc_cache["degree"](dst2).reshape(NC, NPAD // 2, PW)


def _tc2_body(agg_ref, deg_ref, r_ref, b_ref, wl_ref, wr_ref, z_ref, r2_ref):
    s = agg_ref[0] + agg_ref[1]
    d = deg_ref[0] + deg_ref[1]
    e = s * (1.0 / jnp.maximum(d, 1.0)) + b_ref[...] + r_ref[...]
    z_ref[...] = _dot_fast(e, wl_ref[...])
    r2_ref[...] = _dot_fast(e, wr_ref[...])


def _tc2(agg, deg, r1, b, wlbd, wrbd):
    return pl.pallas_call(
        _tc2_body,
        grid=(GRID,),
        in_specs=[
            pl.BlockSpec((NC, HB, PW), lambda i: (0, i, 0)),
            pl.BlockSpec((NC, HB, PW), lambda i: (0, i, 0)),
            pl.BlockSpec((HB, PW), lambda i: (i, 0)),
            pl.BlockSpec((1, PW), lambda i: (0, 0)),
            pl.BlockSpec((PW, PW), lambda i: (0, 0)),
            pl.BlockSpec((PW, PW), lambda i: (0, 0)),
        ],
        out_specs=[
            pl.BlockSpec((HB, PW), lambda i: (i, 0)),
            pl.BlockSpec((HB, PW), lambda i: (i, 0)),
        ],
        out_shape=[
            jax.ShapeDtypeStruct((HN, PW), _f32),
            jax.ShapeDtypeStruct((HN, PW), _f32),
        ],
    )(agg, deg, r1, b, wlbd, wrbd)


def _tc4_body(agg_ref, deg_ref, r_ref, b_ref, ba_ref, bb_ref,
              l1w_ref, l1b_ref, l2w_ref, l2b_ref, l3w_ref, l3b_ref,
              l4w_ref, l4b_ref, out_ref, pooled, cnt):
    i = pl.program_id(0)

    @pl.when(i == 0)
    def _init():
        pooled[...] = jnp.zeros_like(pooled)
        cnt[...] = jnp.zeros_like(cnt)

    s = agg_ref[0] + agg_ref[1]
    d = deg_ref[0] + deg_ref[1]
    e3 = s * (1.0 / jnp.maximum(d, 1.0)) + b_ref[...] + r_ref[...]
    gid = lax.broadcasted_iota(jnp.int32, (HB, G), 1)
    oha = (ba_ref[0, 0, :][:, None] == gid).astype(_f32)
    ohb = (bb_ref[0, 0, :][:, None] == gid).astype(_f32)
    ones = jnp.ones((HB, 1), _f32)
    pooled[...] += _dotT(oha, e3[:, :H]) + _dotT(ohb, e3[:, H:])
    cnt[...] += _dotT(oha, ones) + _dotT(ohb, ones)

    @pl.when(i == GRID - 1)
    def _finish():
        c = pooled[...] * (1.0 / jnp.maximum(cnt[...], 1.0))
        h = jnp.tanh(_dot_fast(c, l1w_ref[...]) + l1b_ref[...])
        h = jnp.tanh(_dot_fast(h, l2w_ref[...]) + l2b_ref[...])
        h = jnp.tanh(_dot_fast(h, l3w_ref[...]) + l3b_ref[...])
        out_ref[...] = _dot_fast(h, l4w_ref[...]) + l4b_ref[...]


def _tc4(agg, deg, r3, b, batch_r, l1w, l1b, l2w, l2b, l3w, l3b, l4w, l4b):
    full = lambda a: pl.BlockSpec(a.shape, lambda i: tuple(0 for _ in a.shape))
    return pl.pallas_call(
        _tc4_body,
        grid=(GRID,),
        in_specs=[
            pl.BlockSpec((NC, HB, PW), lambda i: (0, i, 0)),
            pl.BlockSpec((NC, HB, PW), lambda i: (0, i, 0)),
            pl.BlockSpec((HB, PW), lambda i: (i, 0)),
            pl.BlockSpec((1, PW), lambda i: (0, 0)),
            pl.BlockSpec((1, 1, HB), lambda i: (i, 0, 0)),
            pl.BlockSpec((1, 1, HB), lambda i: (GRID + i, 0, 0)),
            full(l1w), full(l1b), full(l2w), full(l2b),
            full(l3w), full(l3b), full(l4w), full(l4b),
        ],
        out_specs=pl.BlockSpec((G, 80), lambda i: (0, 0)),
        out_shape=jax.ShapeDtypeStruct((G, 80), _f32),
        scratch_shapes=[
            pltpu.VMEM((G, H), _f32),
            pltpu.VMEM((G, 1), _f32),
        ],
    )(agg, deg, r3, b, batch_r, batch_r,
      l1w, l1b, l2w, l2b, l3w, l3b, l4w, l4b)


def kernel(x, edge_index, batch, y, W1l, b1l, W1r, W2l, b2l, W2r, W3l, b3l,
           W3r, lin1_W, lin1_b, bn1_g, bn1_b, bn1_m, bn1_v, lin2_W, lin2_b,
           bn2_g, bn2_b, bn2_m, bn2_v, lin3_W, lin3_b, bn3_g, bn3_b, bn3_m,
           bn3_v, lin4_W, lin4_b):
    rho = lambda v: jnp.where(v < HN, 2 * v, 2 * (v - HN) + 1)
    src = rho(edge_index[0])
    dst = rho(edge_index[1])
    pad = EPAD - E
    pad_i = jnp.arange(pad, dtype=jnp.int32)
    src2 = jnp.concatenate([src, pad_i % N]).reshape(EPAD // CHUNK, CHUNK)
    dst2 = jnp.concatenate([dst, N + pad_i % (NPAD - N)]).reshape(
        EPAD // CHUNK, CHUNK)
    batch_r = batch.reshape(2 * GRID, 1, HB)

    bd = lambda w: jnp.zeros((PW, PW), _f32).at[:H, :H].set(
        w.T).at[H:, H:].set(w.T)
    pk = lambda v: jnp.concatenate([v, v]).reshape(1, PW)
    row = lambda v: v.reshape(1, -1)

    def fold(Wt, b, g, bb, m, v):
        s = g / jnp.sqrt(v + 1e-5)
        return Wt * s[None, :], row(b * s + bb - m * s)

    l1w, l1b = fold(lin1_W.T, lin1_b, bn1_g, bn1_b, bn1_m, bn1_v)
    l2w, l2b = fold(lin2_W.T, lin2_b, bn2_g, bn2_b, bn2_m, bn2_v)
    l3w, l3b = fold(lin3_W.T, lin3_b, bn3_g, bn3_b, bn3_m, bn3_v)
    l4w, l4b = lin4_W.T, row(lin4_b)

    deg = _sc_degree_impl(dst2)
    z1, r1 = _tc1(x, W1l.T, W1r.T)
    agg1 = _sc_scatter_impl(z1, src2, dst2)
    z2, r2 = _tc2(agg1, deg, r1, pk(b1l), bd(W2l), bd(W2r))
    agg2 = _sc_scatter_impl(z2, src2, dst2)
    z3, r3 = _tc2(agg2, deg, r2, pk(b2l), bd(W3l), bd(W3r))
    agg3 = _sc_scatter_impl(z3, src2, dst2)
    return _tc4(agg3, deg, r3, pk(b3l), batch_r,
                l1w, l1b, l2w, l2b, l3w, l3b, l4w, l4b)


# speedup vs baseline: 1.3122x; 1.0121x over previous
import functools

import jax
import jax.numpy as jnp
from jax import lax
from jax.experimental import pallas as pl
from jax.experimental.pallas import tpu as pltpu
from jax.experimental.pallas import tpu_sc as plsc

N = 10000
HN = N // 2
E = 160000
G = 64
F = 500
H = 64
PW = 2 * H
NPAD = 10240
NC, NS = 2, 16
EPAD = 163840
CPT = 40
CHUNK = 128
HB = 1000
GRID = HN // HB
NBUF = 8
ZR = 16
RPT = NPAD // NS

_f32 = jnp.float32


def _dotT(a, b):
    return jax.lax.dot_general(a, b, (((0,), (0,)), ((), ())),
                               preferred_element_type=_f32)


def _dot(a, b):
    return jax.lax.dot_general(a, b, (((1,), (0,)), ((), ())),
                               preferred_element_type=_f32)


def _tc1_body(xa_ref, xb_ref, wlt_ref, wrt_ref, z_ref, r_ref):
    outs = []
    for xref in (xa_ref, xb_ref):
        xb = xref[...]
        nrm = jnp.maximum(jnp.sum(jnp.abs(xb), axis=1, keepdims=True), 1e-12)
        xn = xb / nrm
        outs.append((_dot(xn, wlt_ref[...]), _dot(xn, wrt_ref[...])))
    z_ref[...] = jnp.concatenate([outs[0][0], outs[1][0]], axis=1)
    r_ref[...] = jnp.concatenate([outs[0][1], outs[1][1]], axis=1)


def _tc1(x, wlt, wrt):
    return pl.pallas_call(
        _tc1_body,
        grid=(GRID,),
        in_specs=[
            pl.BlockSpec((HB, F), lambda i: (i, 0)),
            pl.BlockSpec((HB, F), lambda i: (GRID + i, 0)),
            pl.BlockSpec((F, H), lambda i: (0, 0)),
            pl.BlockSpec((F, H), lambda i: (0, 0)),
        ],
        out_specs=[
            pl.BlockSpec((HB, PW), lambda i: (i, 0)),
            pl.BlockSpec((HB, PW), lambda i: (i, 0)),
        ],
        out_shape=[
            jax.ShapeDtypeStruct((HN, PW), _f32),
            jax.ShapeDtypeStruct((HN, PW), _f32),
        ],
    )(x, x, wlt, wrt)


_sc_mesh = dict(core_axis_name="c", subcore_axis_name="s",
                num_cores=NC, num_subcores=NS)


def _sc_zero_acc(sid, acc, zbuf):
    def zrow(i, c):
        for j in range(H // 16):
            zbuf[i, pl.ds(j * 16, 16)] = jnp.zeros((16,), _f32)
        return c
    lax.fori_loop(0, ZR, zrow, 0)

    def zcp(k, c):
        pltpu.sync_copy(zbuf, acc.at[pl.ds(sid * RPT + k * ZR, ZR)])
        return c
    lax.fori_loop(0, RPT // ZR, zcp, 0)


def _make_sc_scatter():
    mesh = plsc.VectorSubcoreMesh(**_sc_mesh)

    @functools.partial(
        pl.kernel,
        out_type=jax.ShapeDtypeStruct((NC * NPAD, H), _f32),
        mesh=mesh,
        compiler_params=pltpu.CompilerParams(use_tc_tiling_on_sc=False),
        scratch_types=[
            pltpu.VMEM_SHARED((NPAD, H), _f32),
            pltpu.VMEM((CPT, CHUNK), jnp.int32),
            pltpu.VMEM((CPT, CHUNK), jnp.int32),
            pltpu.VMEM((NBUF, CHUNK, H), _f32),
            pltpu.VMEM((ZR, H), _f32),
            [pltpu.SemaphoreType.DMA] * NBUF,
        ],
    )
    def sc_fn(z_hbm, src_hbm, dst_hbm, out_hbm, acc, idx_s, idx_d, rows,
              zbuf, sems):
        cid = lax.axis_index("c")
        sid = lax.axis_index("s")
        wid = cid * NS + sid
        _sc_zero_acc(sid, acc, zbuf)

        pltpu.sync_copy(src_hbm.at[pl.ds(wid * CPT, CPT)], idx_s)
        pltpu.sync_copy(dst_hbm.at[pl.ds(wid * CPT, CPT)], idx_d)
        plsc.subcore_barrier()

        def superchunk(s, carry):
            base = s * NBUF
            descs = [
                pltpu.async_copy(z_hbm.at[idx_s.at[base + b]], rows.at[b],
                                 sems[b])
                for b in range(NBUF)
            ]
            for b in range(NBUF):
                descs[b].wait()
                pltpu.sync_copy(rows.at[b], acc.at[idx_d.at[base + b]],
                                add=True)
            return carry
        lax.fori_loop(0, CPT // NBUF, superchunk, 0)
        plsc.subcore_barrier()

        pltpu.sync_copy(acc.at[pl.ds(sid * RPT, RPT)],
                        out_hbm.at[pl.ds(cid * NPAD + sid * RPT, RPT)])

    return sc_fn


def _make_sc_degree():
    mesh = plsc.VectorSubcoreMesh(**_sc_mesh)

    @functools.partial(
        pl.kernel,
        out_type=jax.ShapeDtypeStruct((NC * NPAD, H), _f32),
        mesh=mesh,
        compiler_params=pltpu.CompilerParams(use_tc_tiling_on_sc=False),
        scratch_types=[
            pltpu.VMEM_SHARED((NPAD, H), _f32),
            pltpu.VMEM((CPT, CHUNK), jnp.int32),
            pltpu.VMEM((CHUNK, H), _f32),
            pltpu.VMEM((ZR, H), _f32),
        ],
    )
    def deg_fn(dst_hbm, out_hbm, acc, idx_d, ones, zbuf):
        cid = lax.axis_index("c")
        sid = lax.axis_index("s")
        wid = cid * NS + sid
        _sc_zero_acc(sid, acc, zbuf)

        def orow(i, c):
            for j in range(H // 16):
                ones[i, pl.ds(j * 16, 16)] = jnp.ones((16,), _f32)
            return c
        lax.fori_loop(0, CHUNK, orow, 0)

        pltpu.sync_copy(dst_hbm.at[pl.ds(wid * CPT, CPT)], idx_d)
        plsc.subcore_barrier()

        def chunk(c, carry):
            pltpu.sync_copy(ones, acc.at[idx_d.at[c]], add=True)
            return carry
        lax.fori_loop(0, CPT, chunk, 0)
        plsc.subcore_barrier()

        pltpu.sync_copy(acc.at[pl.ds(sid * RPT, RPT)],
                        out_hbm.at[pl.ds(cid * NPAD + sid * RPT, RPT)])

    return deg_fn


_sc_cache = {}


def _sc_scatter_impl(zp, src2, dst2):
    if "scatter" not in _sc_cache:
        _sc_cache["scatter"] = _make_sc_scatter()
    out = _sc_cache["scatter"](zp.reshape(N, H), src2, dst2)
    return out.reshape(NC, NPAD // 2, PW)


def _sc_degree_impl(dst2):
    if "degree" not in _sc_cache:
        _sc_cache["degree"] = _make_sc_degree()
    return _sc_cache["degree"](dst2).reshape(NC, NPAD // 2, PW)


def _tc2_body(agg_ref, deg_ref, r_ref, b_ref, wl_ref, wr_ref, z_ref, r2_ref):
    s = agg_ref[0] + agg_ref[1]
    d = deg_ref[0] + deg_ref[1]
    e = s * (1.0 / jnp.maximum(d, 1.0)) + b_ref[...] + r_ref[...]
    z_ref[...] = _dot(e, wl_ref[...])
    r2_ref[...] = _dot(e, wr_ref[...])


def _tc2(agg, deg, r1, b, wlbd, wrbd):
    return pl.pallas_call(
        _tc2_body,
        grid=(GRID,),
        in_specs=[
            pl.BlockSpec((NC, HB, PW), lambda i: (0, i, 0)),
            pl.BlockSpec((NC, HB, PW), lambda i: (0, i, 0)),
            pl.BlockSpec((HB, PW), lambda i: (i, 0)),
            pl.BlockSpec((1, PW), lambda i: (0, 0)),
            pl.BlockSpec((PW, PW), lambda i: (0, 0)),
            pl.BlockSpec((PW, PW), lambda i: (0, 0)),
        ],
        out_specs=[
            pl.BlockSpec((HB, PW), lambda i: (i, 0)),
            pl.BlockSpec((HB, PW), lambda i: (i, 0)),
        ],
        out_shape=[
            jax.ShapeDtypeStruct((HN, PW), _f32),
            jax.ShapeDtypeStruct((HN, PW), _f32),
        ],
    )(agg, deg, r1, b, wlbd, wrbd)


def _tc4_body(agg_ref, deg_ref, r_ref, b_ref, ba_ref, bb_ref,
              l1w_ref, l1b_ref, l2w_ref, l2b_ref, l3w_ref, l3b_ref,
              l4w_ref, l4b_ref, out_ref, pooled, cnt):
    i = pl.program_id(0)

    @pl.when(i == 0)
    def _init():
        pooled[...] = jnp.zeros_like(pooled)
        cnt[...] = jnp.zeros_like(cnt)

    s = agg_ref[0] + agg_ref[1]
    d = deg_ref[0] + deg_ref[1]
    e3 = s * (1.0 / jnp.maximum(d, 1.0)) + b_ref[...] + r_ref[...]
    gid = lax.broadcasted_iota(jnp.int32, (HB, G), 1)
    oha = (ba_ref[0, 0, :][:, None] == gid).astype(_f32)
    ohb = (bb_ref[0, 0, :][:, None] == gid).astype(_f32)
    ones = jnp.ones((HB, 1), _f32)
    pooled[...] += _dotT(oha, e3[:, :H]) + _dotT(ohb, e3[:, H:])
    cnt[...] += _dotT(oha, ones) + _dotT(ohb, ones)

    @pl.when(i == GRID - 1)
    def _finish():
        c = pooled[...] * (1.0 / jnp.maximum(cnt[...], 1.0))
        h = jnp.tanh(_dot(c, l1w_ref[...]) + l1b_ref[...])
        h = jnp.tanh(_dot(h, l2w_ref[...]) + l2b_ref[...])
        h = jnp.tanh(_dot(h, l3w_ref[...]) + l3b_ref[...])
        out_ref[...] = _dot(h, l4w_ref[...]) + l4b_ref[...]


def _tc4(agg, deg, r3, b, batch_r, l1w, l1b, l2w, l2b, l3w, l3b, l4w, l4b):
    full = lambda a: pl.BlockSpec(a.shape, lambda i: tuple(0 for _ in a.shape))
    return pl.pallas_call(
        _tc4_body,
        grid=(GRID,),
        in_specs=[
            pl.BlockSpec((NC, HB, PW), lambda i: (0, i, 0)),
            pl.BlockSpec((NC, HB, PW), lambda i: (0, i, 0)),
            pl.BlockSpec((HB, PW), lambda i: (i, 0)),
            pl.BlockSpec((1, PW), lambda i: (0, 0)),
            pl.BlockSpec((1, 1, HB), lambda i: (i, 0, 0)),
            pl.BlockSpec((1, 1, HB), lambda i: (GRID + i, 0, 0)),
            full(l1w), full(l1b), full(l2w), full(l2b),
            full(l3w), full(l3b), full(l4w), full(l4b),
        ],
        out_specs=pl.BlockSpec((G, 80), lambda i: (0, 0)),
        out_shape=jax.ShapeDtypeStruct((G, 80), _f32),
        scratch_shapes=[
            pltpu.VMEM((G, H), _f32),
            pltpu.VMEM((G, 1), _f32),
        ],
    )(agg, deg, r3, b, batch_r, batch_r,
      l1w, l1b, l2w, l2b, l3w, l3b, l4w, l4b)


def kernel(x, edge_index, batch, y, W1l, b1l, W1r, W2l, b2l, W2r, W3l, b3l,
           W3r, lin1_W, lin1_b, bn1_g, bn1_b, bn1_m, bn1_v, lin2_W, lin2_b,
           bn2_g, bn2_b, bn2_m, bn2_v, lin3_W, lin3_b, bn3_g, bn3_b, bn3_m,
           bn3_v, lin4_W, lin4_b):
    rho = lambda v: jnp.where(v < HN, 2 * v, 2 * (v - HN) + 1)
    src = rho(edge_index[0])
    dst = rho(edge_index[1])
    pad = EPAD - E
    pad_i = jnp.arange(pad, dtype=jnp.int32)
    src2 = jnp.concatenate([src, pad_i % N]).reshape(EPAD // CHUNK, CHUNK)
    dst2 = jnp.concatenate([dst, N + pad_i % (NPAD - N)]).reshape(
        EPAD // CHUNK, CHUNK)
    batch_r = batch.reshape(2 * GRID, 1, HB)

    bd = lambda w: jnp.zeros((PW, PW), _f32).at[:H, :H].set(
        w.T).at[H:, H:].set(w.T)
    pk = lambda v: jnp.concatenate([v, v]).reshape(1, PW)
    row = lambda v: v.reshape(1, -1)

    def fold(Wt, b, g, bb, m, v):
        s = g / jnp.sqrt(v + 1e-5)
        return Wt * s[None, :], row(b * s + bb - m * s)

    l1w, l1b = fold(lin1_W.T, lin1_b, bn1_g, bn1_b, bn1_m, bn1_v)
    l2w, l2b = fold(lin2_W.T, lin2_b, bn2_g, bn2_b, bn2_m, bn2_v)
    l3w, l3b = fold(lin3_W.T, lin3_b, bn3_g, bn3_b, bn3_m, bn3_v)
    l4w, l4b = lin4_W.T, row(lin4_b)

    deg = _sc_degree_impl(dst2)
    z1, r1 = _tc1(x, W1l.T, W1r.T)
    agg1 = _sc_scatter_impl(z1, src2, dst2)
    z2, r2 = _tc2(agg1, deg, r1, pk(b1l), bd(W2l), bd(W2r))
    agg2 = _sc_scatter_impl(z2, src2, dst2)
    z3, r3 = _tc2(agg2, deg, r2, pk(b2l), bd(W3l), bd(W3r))
    agg3 = _sc_scatter_impl(z3, src2, dst2)
    return _tc4(agg3, deg, r3, pk(b3l), batch_r,
                l1w, l1b, l2w, l2b, l3w, l3b, l4w, l4b)
